# baseline (speedup 1.0000x reference)
import functools

import jax
import jax.numpy as jnp
from jax import lax
from jax.experimental import pallas as pl
from jax.experimental.pallas import tpu as pltpu
from jax.experimental.pallas import tpu_sc as plsc

_B = 8
_D = 64
_K = 1024


def _conv_body(x_ref, w_ref, b_ref, o_ref, *, taps, H, W, Cin, Cout, relu):
    x = x_ref[0]
    acc = jnp.zeros((H * W, Cout), jnp.float32)
    for t, (dy, dx) in enumerate(taps):
        xs = x[dy:dy + H, dx:dx + W, :].reshape(H * W, Cin)
        acc = acc + jnp.dot(xs, w_ref[t], preferred_element_type=jnp.float32)
    y = acc + b_ref[0]
    if relu:
        y = jnp.maximum(y, 0.0)
    o_ref[0] = y.reshape(H, W, Cout)


def _conv(x, w, b, taps, H, W, Cin, Cout, relu):
    T = len(taps)
    Hp, Wp = x.shape[1], x.shape[2]
    body = functools.partial(_conv_body, taps=taps, H=H, W=W, Cin=Cin,
                             Cout=Cout, relu=relu)
    return pl.pallas_call(
        body,
        grid=(_B,),
        in_specs=[
            pl.BlockSpec((1, Hp, Wp, Cin), lambda bi: (bi, 0, 0, 0)),
            pl.BlockSpec((T, Cin, Cout), lambda bi: (0, 0, 0)),
            pl.BlockSpec((1, Cout), lambda bi: (0, 0)),
        ],
        out_specs=pl.BlockSpec((1, H, W, Cout), lambda bi: (bi, 0, 0, 0)),
        out_shape=jax.ShapeDtypeStruct((_B, H, W, Cout), jnp.float32),
    )(x, w, b)


_DEC_TAPS = (
    ((0, 0, 0), (0, 1, 2), (1, 0, 6), (1, 1, 8)),
    ((0, 1, 1), (1, 1, 7)),
    ((1, 0, 3), (1, 1, 5)),
    ((1, 1, 4),),
)


def _dect_body(*refs, H, Cin, Cout, ste):
    if ste:
        ze_ref, zq_ref, w_ref, b_ref, o_ref = refs
        ze = ze_ref[0]
        x = ze + (zq_ref[0] - ze)
    else:
        x_ref, w_ref, b_ref, o_ref = refs
        x = x_ref[0]
    for p, taps in enumerate(_DEC_TAPS):
        acc = jnp.zeros((H * H, Cout), jnp.float32)
        for (dy, dx, k) in taps:
            xs = x[dy:dy + H, dx:dx + H, :].reshape(H * H, Cin)
            acc = acc + jnp.dot(xs, w_ref[k], preferred_element_type=jnp.float32)
        y = jnp.maximum(acc + b_ref[0], 0.0)
        o_ref[0, :, :, p * Cout:(p + 1) * Cout] = y.reshape(H, H, Cout)


def _dect(xs_list, w9, b, H, Cin, Cout, ste):
    Hp = H + 1
    body = functools.partial(_dect_body, H=H, Cin=Cin, Cout=Cout, ste=ste)
    x_spec = pl.BlockSpec((1, Hp, Hp, Cin), lambda bi: (bi, 0, 0, 0))
    return pl.pallas_call(
        body,
        grid=(_B,),
        in_specs=[x_spec] * len(xs_list) + [
            pl.BlockSpec((9, Cin, Cout), lambda bi: (0, 0, 0)),
            pl.BlockSpec((1, Cout), lambda bi: (0, 0)),
        ],
        out_specs=pl.BlockSpec((1, H, H, 4 * Cout), lambda bi: (bi, 0, 0, 0)),
        out_shape=jax.ShapeDtypeStruct((_B, H, H, 4 * Cout), jnp.float32),
    )(*xs_list, w9, b)


def _dec3_body(x_ref, w_ref, b_ref, o_ref):
    acc = jnp.full((224, 224), b_ref[0], jnp.float32)
    for ci in range(16):
        xc = x_ref[0, ci]
        for dy in range(3):
            for dx in range(3):
                acc = acc + xc[dy:dy + 224, dx:dx + 224] * w_ref[3 * dy + dx, ci]
    o_ref[0] = acc


def _dec3(xp, w, b):
    return pl.pallas_call(
        _dec3_body,
        grid=(_B,),
        in_specs=[
            pl.BlockSpec((1, 16, 226, 226), lambda bi: (bi, 0, 0, 0)),
            pl.BlockSpec(memory_space=pltpu.SMEM),
            pl.BlockSpec(memory_space=pltpu.SMEM),
        ],
        out_specs=pl.BlockSpec((1, 224, 224), lambda bi: (bi, 0, 0)),
        out_shape=jax.ShapeDtypeStruct((_B, 224, 224), jnp.float32),
    )(xp, w, b)


_VQ_R = 1568
_VQ_G = (_B * 56 * 56) // _VQ_R


def _vq_body(z_ref, cbt_ref, c2_ref, o_ref):
    z = z_ref[...]
    s = c2_ref[...] - 2.0 * jnp.dot(z, cbt_ref[...],
                                    preferred_element_type=jnp.float32)
    m = jnp.min(s, axis=1, keepdims=True)
    lane = lax.broadcasted_iota(jnp.int32, s.shape, 1)
    idx = jnp.min(jnp.where(s == m, lane, _K), axis=1)
    o_ref[0, 0] = idx


def _vq(zf, cbt, c2):
    out = pl.pallas_call(
        _vq_body,
        grid=(_VQ_G,),
        in_specs=[
            pl.BlockSpec((_VQ_R, _D), lambda i: (i, 0)),
            pl.BlockSpec((_D, _K), lambda i: (0, 0)),
            pl.BlockSpec((1, _K), lambda i: (0, 0)),
        ],
        out_specs=pl.BlockSpec((1, 1, _VQ_R), lambda i: (i, 0, 0)),
        out_shape=jax.ShapeDtypeStruct((_VQ_G, 1, _VQ_R), jnp.int32),
    )(zf, cbt, c2)
    return out.reshape(-1)


_NC, _NS = 2, 16
_NW = _NC * _NS
_ROWS = _B * 56 * 56
_RPW = _ROWS // _NW


def _sc_gather_body(idx_hbm, table_hbm, out_hbm, idx_v, rows_v, sem):
    wid = lax.axis_index("s") * _NC + lax.axis_index("c")
    base = wid * _RPW
    pltpu.sync_copy(idx_hbm.at[pl.ds(base, _RPW)], idx_v)
    pltpu.async_copy(table_hbm.at[idx_v], rows_v, sem).wait()
    pltpu.sync_copy(rows_v, out_hbm.at[pl.ds(base, _RPW)])


def _sc_gather(idx, table128):
    mesh = plsc.VectorSubcoreMesh(core_axis_name="c", subcore_axis_name="s")
    f = functools.partial(
        pl.kernel,
        mesh=mesh,
        out_type=jax.ShapeDtypeStruct((_ROWS, 128), jnp.float32),
        scratch_types=[
            pltpu.VMEM((_RPW,), jnp.int32),
            pltpu.VMEM((_RPW, 128), jnp.float32),
            pltpu.SemaphoreType.DMA,
        ],
    )(_sc_gather_body)
    return f(idx, table128)


_TAPS4 = ((0, 0), (0, 1), (1, 0), (1, 1))
_TAPS9 = tuple((dy, dx) for dy in range(3) for dx in range(3))


def _s2d(x):
    B, H, W, C = x.shape
    s = (x.reshape(B, H // 2, 2, W // 2, 2, C)
         .transpose(0, 1, 3, 2, 4, 5)
         .reshape(B, H // 2, W // 2, 4 * C))
    return jnp.pad(s, ((0, 0), (0, 1), (0, 1), (0, 0)))


def _s2d_weight(w):
    _, _, ci, co = w.shape
    wp = jnp.pad(w, ((0, 1), (0, 1), (0, 0), (0, 0)))
    w4 = wp.reshape(2, 2, 2, 2, ci, co).transpose(0, 2, 1, 3, 4, 5)
    return w4.reshape(4, 4 * ci, co)


def _d2s(y, Cout):
    B, H, W, _ = y.shape
    return (y.reshape(B, H, W, 2, 2, Cout)
            .transpose(0, 1, 3, 2, 4, 5)
            .reshape(B, 2 * H, 2 * W, Cout))


def kernel(inputs, enc_w1, enc_b1, enc_w2, enc_b2, enc_w3, enc_b3, codebook,
           dec_w1, dec_b1, dec_w2, dec_b2, dec_w3, dec_b3):
    x1 = _s2d(inputs)
    h1 = _conv(x1, _s2d_weight(enc_w1), enc_b1[None], _TAPS4,
               112, 112, 4, 16, relu=True)
    x2 = _s2d(h1)
    h2 = _conv(x2, _s2d_weight(enc_w2), enc_b2[None], _TAPS4,
               56, 56, 64, 32, relu=True)
    x3 = jnp.pad(h2, ((0, 0), (1, 1), (1, 1), (0, 0)))
    z_e = _conv(x3, enc_w3.reshape(9, 32, 64), enc_b3[None], _TAPS9,
                56, 56, 32, 64, relu=False)

    zf = z_e.reshape(_ROWS, _D)
    cbt = codebook.T
    c2 = jnp.sum(codebook * codebook, axis=1)[None, :]
    qzf = _vq(zf, cbt, c2)
    q_z = qzf.reshape(_B, 56, 56)
    cb128 = jnp.pad(codebook, ((0, 0), (0, 128 - _D)))
    zq_f = _sc_gather(qzf, cb128)
    z_q = zq_f[:, :_D].reshape(_B, 56, 56, _D)

    ze_p = jnp.pad(z_e, ((0, 0), (1, 0), (1, 0), (0, 0)))
    zq_p = jnp.pad(z_q, ((0, 0), (1, 0), (1, 0), (0, 0)))
    g1 = _dect([ze_p, zq_p], dec_w1.reshape(9, 64, 32), dec_b1[None],
               56, 64, 32, ste=True)
    g1i = _d2s(g1, 32)
    g1p = jnp.pad(g1i, ((0, 0), (1, 0), (1, 0), (0, 0)))
    g2 = _dect([g1p], dec_w2.reshape(9, 32, 16), dec_b2[None],
               112, 32, 16, ste=False)
    g2p = (g2.reshape(_B, 112, 112, 2, 2, 16)
           .transpose(0, 5, 1, 3, 2, 4)
           .reshape(_B, 16, 224, 224))
    g2p = jnp.pad(g2p, ((0, 0), (0, 0), (1, 1), (1, 1)))
    logits = _dec3(g2p, dec_w3.reshape(9, 16), dec_b3)

    return (logits[..., None], z_e, z_q, q_z)


# speedup vs baseline: 1.9176x; 1.9176x over previous
import functools

import jax
import jax.numpy as jnp
from jax import lax
from jax.experimental import pallas as pl
from jax.experimental.pallas import tpu as pltpu
from jax.experimental.pallas import tpu_sc as plsc

_B = 8
_D = 64
_K = 1024
_H = 56


def _pad_end(x, n=1):
    return jnp.pad(x, ((0, n), (0, n), (0, 0)))


def _pad_begin(x, n=1):
    return jnp.pad(x, ((n, 0), (n, 0), (0, 0)))


def _enc_body(x_ref, w1_ref, b1_ref, w2_ref, b2_ref, w3_ref, b3_ref, o_ref):
    n = _H * _H
    x = x_ref[0]
    acc = jnp.zeros((n, 64), jnp.float32)
    for t, (oy, ox) in enumerate(((0, 0), (0, 1), (1, 0), (1, 1))):
        xs = x[oy:oy + _H, ox:ox + _H, :].reshape(n, 16)
        acc = acc + jnp.dot(xs, w1_ref[t], preferred_element_type=jnp.float32)
    s1 = jnp.maximum(acc + b1_ref[0], 0.0).reshape(_H, _H, 64)
    s1p = _pad_end(s1)
    acc = jnp.zeros((n, 32), jnp.float32)
    for t, (oy, ox) in enumerate(((0, 0), (0, 1), (1, 0), (1, 1))):
        xs = s1p[oy:oy + _H, ox:ox + _H, :].reshape(n, 64)
        acc = acc + jnp.dot(xs, w2_ref[t], preferred_element_type=jnp.float32)
    s2 = jnp.maximum(acc + b2_ref[0], 0.0).reshape(_H, _H, 32)
    s2p = jnp.pad(s2, ((1, 1), (1, 1), (0, 0)))
    acc = jnp.zeros((n, 64), jnp.float32)
    for t in range(9):
        dy, dx = divmod(t, 3)
        xs = s2p[dy:dy + _H, dx:dx + _H, :].reshape(n, 32)
        acc = acc + jnp.dot(xs, w3_ref[t], preferred_element_type=jnp.float32)
    o_ref[0] = (acc + b3_ref[0]).reshape(_H, _H, 64)


def _encoder(x16, w1, b1, w2, b2, w3, b3):
    return pl.pallas_call(
        _enc_body,
        grid=(_B,),
        in_specs=[
            pl.BlockSpec((1, 57, 57, 16), lambda bi: (bi, 0, 0, 0)),
            pl.BlockSpec((4, 16, 64), lambda bi: (0, 0, 0)),
            pl.BlockSpec((1, 64), lambda bi: (0, 0)),
            pl.BlockSpec((4, 64, 32), lambda bi: (0, 0, 0)),
            pl.BlockSpec((1, 32), lambda bi: (0, 0)),
            pl.BlockSpec((9, 32, 64), lambda bi: (0, 0, 0)),
            pl.BlockSpec((1, 64), lambda bi: (0, 0)),
        ],
        out_specs=pl.BlockSpec((1, _H, _H, 64), lambda bi: (bi, 0, 0, 0)),
        out_shape=jax.ShapeDtypeStruct((_B, _H, _H, 64), jnp.float32),
    )(x16, w1, b1, w2, b2, w3, b3)


_DEC_TAPS = (
    ((0, 0, 0), (0, 1, 2), (1, 0, 6), (1, 1, 8)),
    ((0, 1, 1), (1, 1, 7)),
    ((1, 0, 3), (1, 1, 5)),
    ((1, 1, 4),),
)

_D2_ROW = (
    ((1, 0, 0), (0, 2, 1)),
    ((0, 1, 1),),
    ((0, 0, 1), (1, 2, 1)),
    ((1, 1, 1),),
)


def _dec_body(ze_ref, zq_ref, w1_ref, b1_ref, w2_ref, b2_ref, w3_ref, b3_ref,
              o_ref):
    n = _H * _H
    ze = ze_ref[0]
    x = ze + (zq_ref[0] - ze)
    xp = _pad_begin(x)
    acc = jnp.zeros((n, 128), jnp.float32)
    for t in range(4):
        dy, dx = divmod(t, 2)
        xs = xp[dy:dy + _H, dx:dx + _H, :].reshape(n, 64)
        acc = acc + jnp.dot(xs, w1_ref[t], preferred_element_type=jnp.float32)
    g = jnp.maximum(acc + b1_ref[0], 0.0).reshape(_H, _H, 128)
    gp = _pad_begin(g)
    acc = jnp.zeros((n, 256), jnp.float32)
    for t in range(4):
        dy, dx = divmod(t, 2)
        xs = gp[dy:dy + _H, dx:dx + _H, :].reshape(n, 128)
        acc = acc + jnp.dot(xs, w2_ref[t], preferred_element_type=jnp.float32)
    z2 = jnp.maximum(acc + b2_ref[0], 0.0).reshape(_H, _H, 256)
    z2p = jnp.pad(z2, ((1, 1), (1, 1), (0, 0)))
    acc = jnp.zeros((n, 16), jnp.float32)
    for t in range(9):
        sy, sx = divmod(t, 3)
        xs = z2p[sy:sy + _H, sx:sx + _H, :].reshape(n, 256)
        acc = acc + jnp.dot(xs, w3_ref[t], preferred_element_type=jnp.float32)
    o_ref[0] = (acc + b3_ref[0]).reshape(_H, _H, 16)


def _decoder(ze, zq, w1, b1, w2, b2, w3, b3):
    return pl.pallas_call(
        _dec_body,
        grid=(_B,),
        in_specs=[
            pl.BlockSpec((1, _H, _H, 64), lambda bi: (bi, 0, 0, 0)),
            pl.BlockSpec((1, _H, _H, 64), lambda bi: (bi, 0, 0, 0)),
            pl.BlockSpec((4, 64, 128), lambda bi: (0, 0, 0)),
            pl.BlockSpec((1, 128), lambda bi: (0, 0)),
            pl.BlockSpec((4, 128, 256), lambda bi: (0, 0, 0)),
            pl.BlockSpec((1, 256), lambda bi: (0, 0)),
            pl.BlockSpec((9, 256, 16), lambda bi: (0, 0, 0)),
            pl.BlockSpec((1, 16), lambda bi: (0, 0)),
        ],
        out_specs=pl.BlockSpec((1, _H, _H, 16), lambda bi: (bi, 0, 0, 0)),
        out_shape=jax.ShapeDtypeStruct((_B, _H, _H, 16), jnp.float32),
    )(ze, zq, w1, b1, w2, b2, w3, b3)


_VQ_R = 1568
_ROWS = _B * _H * _H
_VQ_G = _ROWS // _VQ_R


def _vq_body(z_ref, cbt_ref, c2_ref, o_ref):
    z = z_ref[...]
    s = c2_ref[...] - 2.0 * jnp.dot(z, cbt_ref[...],
                                    preferred_element_type=jnp.float32)
    m = jnp.min(s, axis=1, keepdims=True)
    lane = lax.broadcasted_iota(jnp.int32, s.shape, 1)
    idx = jnp.min(jnp.where(s == m, lane, _K), axis=1)
    o_ref[0, 0] = idx


def _vq(zf, cbt, c2):
    out = pl.pallas_call(
        _vq_body,
        grid=(_VQ_G,),
        in_specs=[
            pl.BlockSpec((_VQ_R, _D), lambda i: (i, 0)),
            pl.BlockSpec((_D, _K), lambda i: (0, 0)),
            pl.BlockSpec((1, _K), lambda i: (0, 0)),
        ],
        out_specs=pl.BlockSpec((1, 1, _VQ_R), lambda i: (i, 0, 0)),
        out_shape=jax.ShapeDtypeStruct((_VQ_G, 1, _VQ_R), jnp.int32),
    )(zf, cbt, c2)
    return out.reshape(-1)


_NC, _NS = 2, 16
_NW = _NC * _NS
_RPW = _ROWS // _NW


def _sc_gather_body(idx_hbm, table_hbm, out_hbm, idx_v, rows_v, sem):
    wid = lax.axis_index("s") * _NC + lax.axis_index("c")
    base = wid * _RPW
    pltpu.sync_copy(idx_hbm.at[pl.ds(base, _RPW)], idx_v)
    pltpu.async_copy(table_hbm.at[idx_v], rows_v, sem).wait()
    pltpu.sync_copy(rows_v, out_hbm.at[pl.ds(base, _RPW)])


def _sc_gather(idx, table128):
    mesh = plsc.VectorSubcoreMesh(core_axis_name="c", subcore_axis_name="s")
    f = functools.partial(
        pl.kernel,
        mesh=mesh,
        out_type=jax.ShapeDtypeStruct((_ROWS, 128), jnp.float32),
        scratch_types=[
            pltpu.VMEM((_RPW,), jnp.int32),
            pltpu.VMEM((_RPW, 128), jnp.float32),
            pltpu.SemaphoreType.DMA,
        ],
    )(_sc_gather_body)
    return f(idx, table128)


def _s2d16(x):
    s = (x.reshape(_B, _H, 4, _H, 4)
         .transpose(0, 1, 3, 2, 4)
         .reshape(_B, _H, _H, 16))
    return jnp.pad(s, ((0, 0), (0, 1), (0, 1), (0, 0)))


def _enc1_weight(w):
    w1 = jnp.zeros((2, 2, 16, 64), jnp.float32)
    for p in range(2):
        for dy in range(3):
            oy, ry = divmod(2 * p + dy, 4)
            for q in range(2):
                for dx in range(3):
                    ox, rx = divmod(2 * q + dx, 4)
                    w1 = w1.at[oy, ox, 4 * ry + rx,
                               (2 * p + q) * 16:(2 * p + q) * 16 + 16].set(
                                   w[dy, dx, 0, :])
    return w1.reshape(4, 16, 64)


def _dec1_weight(w):
    C = jnp.zeros((2, 2, 64, 128), jnp.float32)
    for p, taps in enumerate(_DEC_TAPS):
        for (dy, dx, k) in taps:
            ky, kx = divmod(k, 3)
            C = C.at[dy, dx, :, p * 32:(p + 1) * 32].add(w[ky, kx])
    return C.reshape(4, 64, 128)


def _dec2_weight(w):
    Bm = jnp.zeros((2, 2, 128, 256), jnp.float32)
    for ry in range(4):
        for (py, ky, oy) in _D2_ROW[ry]:
            for rx in range(4):
                for (px, kx, ox) in _D2_ROW[rx]:
                    si = (2 * py + px) * 32
                    so = (4 * ry + rx) * 16
                    Bm = Bm.at[oy, ox, si:si + 32, so:so + 16].add(w[ky, kx])
    return Bm.reshape(4, 128, 256)


def _dec3_weight(w):
    A = jnp.zeros((3, 3, 256, 16), jnp.float32)
    for ry in range(4):
        for ky in range(3):
            sy, py = divmod(ry + ky - 1, 4)
            for rx in range(4):
                for kx in range(3):
                    sx, px = divmod(rx + kx - 1, 4)
                    src = (4 * py + px) * 16
                    A = A.at[sy + 1, sx + 1, src:src + 16,
                             4 * ry + rx].add(w[ky, kx, :, 0])
    return A.reshape(9, 256, 16)


def _s2d_weight(w):
    _, _, ci, co = w.shape
    wp = jnp.pad(w, ((0, 1), (0, 1), (0, 0), (0, 0)))
    w4 = wp.reshape(2, 2, 2, 2, ci, co).transpose(0, 2, 1, 3, 4, 5)
    return w4.reshape(4, 4 * ci, co)


def kernel(inputs, enc_w1, enc_b1, enc_w2, enc_b2, enc_w3, enc_b3, codebook,
           dec_w1, dec_b1, dec_w2, dec_b2, dec_w3, dec_b3):
    x16 = _s2d16(inputs)
    z_e = _encoder(x16, _enc1_weight(enc_w1), jnp.tile(enc_b1, 4)[None],
                   _s2d_weight(enc_w2), enc_b2[None],
                   enc_w3.reshape(9, 32, 64), enc_b3[None])

    zf = z_e.reshape(_ROWS, _D)
    cbt = codebook.T
    c2 = jnp.sum(codebook * codebook, axis=1)[None, :]
    qzf = _vq(zf, cbt, c2)
    q_z = qzf.reshape(_B, _H, _H)
    cb128 = jnp.pad(codebook, ((0, 0), (0, 128 - _D)))
    zq_f = _sc_gather(qzf, cb128)
    z_q = zq_f[:, :_D].reshape(_B, _H, _H, _D)

    ph = _decoder(z_e, z_q, _dec1_weight(dec_w1), jnp.tile(dec_b1, 4)[None],
                  _dec2_weight(dec_w2), jnp.tile(dec_b2, 16)[None],
                  _dec3_weight(dec_w3), jnp.tile(dec_b3, 16)[None])
    logits = (ph.reshape(_B, _H, _H, 4, 4)
              .transpose(0, 1, 3, 2, 4)
              .reshape(_B, 224, 224, 1))
    return (logits, z_e, z_q, q_z)


# speedup vs baseline: 2.0854x; 1.0875x over previous
import functools

import jax
import jax.numpy as jnp
from jax import lax
from jax.experimental import pallas as pl
from jax.experimental.pallas import tpu as pltpu
from jax.experimental.pallas import tpu_sc as plsc

_B = 8
_D = 64
_K = 1024
_H = 56


def _pad_end(x, n=1):
    return jnp.pad(x, ((0, n), (0, n), (0, 0)))


def _pad_begin(x, n=1):
    return jnp.pad(x, ((n, 0), (n, 0), (0, 0)))


def _enc_body(x_ref, w1_ref, b1_ref, w2_ref, b2_ref, w3_ref, b3_ref, o_ref):
    n = _H * _H
    x = x_ref[0]
    acc = jnp.zeros((n, 64), jnp.float32)
    for t, (oy, ox) in enumerate(((0, 0), (0, 1), (1, 0), (1, 1))):
        xs = x[oy:oy + _H, ox:ox + _H, :].reshape(n, 16)
        acc = acc + jnp.dot(xs, w1_ref[t], preferred_element_type=jnp.float32)
    s1 = jnp.maximum(acc + b1_ref[0], 0.0).reshape(_H, _H, 64)
    s1p = _pad_end(s1)
    acc = jnp.zeros((n, 32), jnp.float32)
    for t, (oy, ox) in enumerate(((0, 0), (0, 1), (1, 0), (1, 1))):
        xs = s1p[oy:oy + _H, ox:ox + _H, :].reshape(n, 64)
        acc = acc + jnp.dot(xs, w2_ref[t], preferred_element_type=jnp.float32)
    s2 = jnp.maximum(acc + b2_ref[0], 0.0).reshape(_H, _H, 32)
    s2p = jnp.pad(s2, ((1, 1), (1, 1), (0, 0)))
    acc = jnp.zeros((n, 64), jnp.float32)
    for t in range(9):
        dy, dx = divmod(t, 3)
        xs = s2p[dy:dy + _H, dx:dx + _H, :].reshape(n, 32)
        acc = acc + jnp.dot(xs, w3_ref[t], preferred_element_type=jnp.float32)
    o_ref[0] = (acc + b3_ref[0]).reshape(_H, _H, 64)


def _encoder(x16, w1, b1, w2, b2, w3, b3):
    return pl.pallas_call(
        _enc_body,
        grid=(_B,),
        in_specs=[
            pl.BlockSpec((1, 57, 57, 16), lambda bi: (bi, 0, 0, 0)),
            pl.BlockSpec((4, 16, 64), lambda bi: (0, 0, 0)),
            pl.BlockSpec((1, 64), lambda bi: (0, 0)),
            pl.BlockSpec((4, 64, 32), lambda bi: (0, 0, 0)),
            pl.BlockSpec((1, 32), lambda bi: (0, 0)),
            pl.BlockSpec((9, 32, 64), lambda bi: (0, 0, 0)),
            pl.BlockSpec((1, 64), lambda bi: (0, 0)),
        ],
        out_specs=pl.BlockSpec((1, _H, _H, 64), lambda bi: (bi, 0, 0, 0)),
        out_shape=jax.ShapeDtypeStruct((_B, _H, _H, 64), jnp.float32),
    )(x16, w1, b1, w2, b2, w3, b3)


_DEC_TAPS = (
    ((0, 0, 0), (0, 1, 2), (1, 0, 6), (1, 1, 8)),
    ((0, 1, 1), (1, 1, 7)),
    ((1, 0, 3), (1, 1, 5)),
    ((1, 1, 4),),
)

_D2_ROW = (
    ((1, 0, 0), (0, 2, 1)),
    ((0, 1, 1),),
    ((0, 0, 1), (1, 2, 1)),
    ((1, 1, 1),),
)


def _dec_body(ze_ref, zq_ref, w1_ref, b1_ref, w2_ref, b2_ref, w3_ref, b3_ref,
              o_ref):
    n = _H * _H
    ze = ze_ref[0]
    x = ze + (zq_ref[0] - ze)
    xp = _pad_begin(x)
    acc = jnp.zeros((n, 128), jnp.float32)
    for t in range(4):
        dy, dx = divmod(t, 2)
        xs = xp[dy:dy + _H, dx:dx + _H, :].reshape(n, 64)
        acc = acc + jnp.dot(xs, w1_ref[t], preferred_element_type=jnp.float32)
    g = jnp.maximum(acc + b1_ref[0], 0.0).reshape(_H, _H, 128)
    gp = _pad_begin(g)
    acc = jnp.zeros((n, 256), jnp.float32)
    for t in range(4):
        dy, dx = divmod(t, 2)
        xs = gp[dy:dy + _H, dx:dx + _H, :].reshape(n, 128)
        acc = acc + jnp.dot(xs, w2_ref[t], preferred_element_type=jnp.float32)
    z2 = jnp.maximum(acc + b2_ref[0], 0.0).reshape(_H, _H, 256)
    z2p = jnp.pad(z2, ((1, 1), (1, 1), (0, 0)))
    acc = jnp.zeros((n, 16), jnp.float32)
    for t in range(9):
        sy, sx = divmod(t, 3)
        xs = z2p[sy:sy + _H, sx:sx + _H, :].reshape(n, 256)
        acc = acc + jnp.dot(xs, w3_ref[t], preferred_element_type=jnp.float32)
    o_ref[0] = (acc + b3_ref[0]).reshape(_H, _H, 16)


def _decoder(ze, zq, w1, b1, w2, b2, w3, b3):
    return pl.pallas_call(
        _dec_body,
        grid=(_B,),
        in_specs=[
            pl.BlockSpec((1, _H, _H, 64), lambda bi: (bi, 0, 0, 0)),
            pl.BlockSpec((1, _H, _H, 64), lambda bi: (bi, 0, 0, 0)),
            pl.BlockSpec((4, 64, 128), lambda bi: (0, 0, 0)),
            pl.BlockSpec((1, 128), lambda bi: (0, 0)),
            pl.BlockSpec((4, 128, 256), lambda bi: (0, 0, 0)),
            pl.BlockSpec((1, 256), lambda bi: (0, 0)),
            pl.BlockSpec((9, 256, 16), lambda bi: (0, 0, 0)),
            pl.BlockSpec((1, 16), lambda bi: (0, 0)),
        ],
        out_specs=pl.BlockSpec((1, _H, _H, 16), lambda bi: (bi, 0, 0, 0)),
        out_shape=jax.ShapeDtypeStruct((_B, _H, _H, 16), jnp.float32),
    )(ze, zq, w1, b1, w2, b2, w3, b3)


_VQ_R = 1568
_ROWS = _B * _H * _H
_VQ_G = _ROWS // _VQ_R


def _vq_body(z_ref, cbt_ref, c2_ref, o_ref):
    z = z_ref[...]
    s = c2_ref[...] - 2.0 * jnp.dot(z, cbt_ref[...],
                                    preferred_element_type=jnp.float32)
    m = jnp.min(s, axis=1, keepdims=True)
    lane = lax.broadcasted_iota(jnp.int32, s.shape, 1)
    idx = jnp.min(jnp.where(s == m, lane, _K), axis=1)
    o_ref[0, 0] = idx


def _vq(zf, cbt, c2):
    out = pl.pallas_call(
        _vq_body,
        grid=(_VQ_G,),
        in_specs=[
            pl.BlockSpec((_VQ_R, _D), lambda i: (i, 0)),
            pl.BlockSpec((_D, _K), lambda i: (0, 0)),
            pl.BlockSpec((1, _K), lambda i: (0, 0)),
        ],
        out_specs=pl.BlockSpec((1, 1, _VQ_R), lambda i: (i, 0, 0)),
        out_shape=jax.ShapeDtypeStruct((_VQ_G, 1, _VQ_R), jnp.int32),
    )(zf, cbt, c2)
    return out.reshape(-1)


_NC, _NS = 2, 16
_NW = _NC * _NS
_RPW = _ROWS // _NW


def _sc_gather_body(idx_hbm, table_hbm, out_hbm, idx_v, rows_v, tbl_sh, sem):
    s = lax.axis_index("s")
    wid = s * _NC + lax.axis_index("c")
    base = wid * _RPW
    pltpu.sync_copy(idx_hbm.at[pl.ds(base, _RPW)], idx_v)

    @pl.when(s == 0)
    def _stage_table():
        pltpu.sync_copy(table_hbm, tbl_sh)

    plsc.subcore_barrier()
    pltpu.async_copy(tbl_sh.at[idx_v], rows_v, sem).wait()
    pltpu.sync_copy(rows_v, out_hbm.at[pl.ds(base, _RPW)])


def _sc_gather(idx, table128):
    mesh = plsc.VectorSubcoreMesh(core_axis_name="c", subcore_axis_name="s")
    f = functools.partial(
        pl.kernel,
        mesh=mesh,
        out_type=jax.ShapeDtypeStruct((_ROWS, 128), jnp.float32),
        scratch_types=[
            pltpu.VMEM((_RPW,), jnp.int32),
            pltpu.VMEM((_RPW, 128), jnp.float32),
            pltpu.VMEM_SHARED((_K, 128), jnp.float32),
            pltpu.SemaphoreType.DMA,
        ],
    )(_sc_gather_body)
    return f(idx, table128)


def _s2d16(x):
    s = (x.reshape(_B, _H, 4, _H, 4)
         .transpose(0, 1, 3, 2, 4)
         .reshape(_B, _H, _H, 16))
    return jnp.pad(s, ((0, 0), (0, 1), (0, 1), (0, 0)))


def _enc1_weight(w):
    w1 = jnp.zeros((2, 2, 16, 64), jnp.float32)
    for p in range(2):
        for dy in range(3):
            oy, ry = divmod(2 * p + dy, 4)
            for q in range(2):
                for dx in range(3):
                    ox, rx = divmod(2 * q + dx, 4)
                    w1 = w1.at[oy, ox, 4 * ry + rx,
                               (2 * p + q) * 16:(2 * p + q) * 16 + 16].set(
                                   w[dy, dx, 0, :])
    return w1.reshape(4, 16, 64)


def _dec1_weight(w):
    C = jnp.zeros((2, 2, 64, 128), jnp.float32)
    for p, taps in enumerate(_DEC_TAPS):
        for (dy, dx, k) in taps:
            ky, kx = divmod(k, 3)
            C = C.at[dy, dx, :, p * 32:(p + 1) * 32].add(w[ky, kx])
    return C.reshape(4, 64, 128)


def _dec2_weight(w):
    Bm = jnp.zeros((2, 2, 128, 256), jnp.float32)
    for ry in range(4):
        for (py, ky, oy) in _D2_ROW[ry]:
            for rx in range(4):
                for (px, kx, ox) in _D2_ROW[rx]:
                    si = (2 * py + px) * 32
                    so = (4 * ry + rx) * 16
                    Bm = Bm.at[oy, ox, si:si + 32, so:so + 16].add(w[ky, kx])
    return Bm.reshape(4, 128, 256)


def _dec3_weight(w):
    A = jnp.zeros((3, 3, 256, 16), jnp.float32)
    for ry in range(4):
        for ky in range(3):
            sy, py = divmod(ry + ky - 1, 4)
            for rx in range(4):
                for kx in range(3):
                    sx, px = divmod(rx + kx - 1, 4)
                    src = (4 * py + px) * 16
                    A = A.at[sy + 1, sx + 1, src:src + 16,
                             4 * ry + rx].add(w[ky, kx, :, 0])
    return A.reshape(9, 256, 16)


def _s2d_weight(w):
    _, _, ci, co = w.shape
    wp = jnp.pad(w, ((0, 1), (0, 1), (0, 0), (0, 0)))
    w4 = wp.reshape(2, 2, 2, 2, ci, co).transpose(0, 2, 1, 3, 4, 5)
    return w4.reshape(4, 4 * ci, co)


def kernel(inputs, enc_w1, enc_b1, enc_w2, enc_b2, enc_w3, enc_b3, codebook,
           dec_w1, dec_b1, dec_w2, dec_b2, dec_w3, dec_b3):
    x16 = _s2d16(inputs)
    z_e = _encoder(x16, _enc1_weight(enc_w1), jnp.tile(enc_b1, 4)[None],
                   _s2d_weight(enc_w2), enc_b2[None],
                   enc_w3.reshape(9, 32, 64), enc_b3[None])

    zf = z_e.reshape(_ROWS, _D)
    cbt = codebook.T
    c2 = jnp.sum(codebook * codebook, axis=1)[None, :]
    qzf = _vq(zf, cbt, c2)
    q_z = qzf.reshape(_B, _H, _H)
    cb128 = jnp.pad(codebook, ((0, 0), (0, 128 - _D)))
    zq_f = _sc_gather(qzf, cb128)
    z_q = zq_f[:, :_D].reshape(_B, _H, _H, _D)

    ph = _decoder(z_e, z_q, _dec1_weight(dec_w1), jnp.tile(dec_b1, 4)[None],
                  _dec2_weight(dec_w2), jnp.tile(dec_b2, 16)[None],
                  _dec3_weight(dec_w3), jnp.tile(dec_b3, 16)[None])
    logits = (ph.reshape(_B, _H, _H, 4, 4)
              .transpose(0, 1, 3, 2, 4)
              .reshape(_B, 224, 224, 1))
    return (logits, z_e, z_q, q_z)


# speedup vs baseline: 2.5508x; 1.2232x over previous
import functools

import jax
import jax.numpy as jnp
import numpy as np
from jax import lax
from jax.experimental import pallas as pl
from jax.experimental.pallas import tpu as pltpu
from jax.experimental.pallas import tpu_sc as plsc

_B = 8
_D = 64
_K = 1024
_H = 56


def _pad_end(x, n=1):
    return jnp.pad(x, ((0, n), (0, n), (0, 0)))


def _pad_begin(x, n=1):
    return jnp.pad(x, ((n, 0), (n, 0), (0, 0)))


def _enc_body(x_ref, w1_ref, b1_ref, w2_ref, b2_ref, w3_ref, b3_ref, o_ref):
    n = _H * _H
    x = x_ref[0]
    acc = jnp.zeros((n, 64), jnp.float32)
    for t, (oy, ox) in enumerate(((0, 0), (0, 1), (1, 0), (1, 1))):
        xs = x[oy:oy + _H, ox:ox + _H, :].reshape(n, 16)
        acc = acc + jnp.dot(xs, w1_ref[t], preferred_element_type=jnp.float32)
    s1 = jnp.maximum(acc + b1_ref[0], 0.0).reshape(_H, _H, 64)
    s1p = _pad_end(s1)
    acc = jnp.zeros((n, 32), jnp.float32)
    for t, (oy, ox) in enumerate(((0, 0), (0, 1), (1, 0), (1, 1))):
        xs = s1p[oy:oy + _H, ox:ox + _H, :].reshape(n, 64)
        acc = acc + jnp.dot(xs, w2_ref[t], preferred_element_type=jnp.float32)
    s2 = jnp.maximum(acc + b2_ref[0], 0.0).reshape(_H, _H, 32)
    s2p = jnp.pad(s2, ((1, 1), (1, 1), (0, 0)))
    acc = jnp.zeros((n, 64), jnp.float32)
    for t in range(9):
        dy, dx = divmod(t, 3)
        xs = s2p[dy:dy + _H, dx:dx + _H, :].reshape(n, 32)
        acc = acc + jnp.dot(xs, w3_ref[t], preferred_element_type=jnp.float32)
    o_ref[0] = (acc + b3_ref[0]).reshape(_H, _H, 64)


def _encoder(x16, w1, b1, w2, b2, w3, b3):
    return pl.pallas_call(
        _enc_body,
        grid=(_B,),
        in_specs=[
            pl.BlockSpec((1, 57, 57, 16), lambda bi: (bi, 0, 0, 0)),
            pl.BlockSpec((4, 16, 64), lambda bi: (0, 0, 0)),
            pl.BlockSpec((1, 64), lambda bi: (0, 0)),
            pl.BlockSpec((4, 64, 32), lambda bi: (0, 0, 0)),
            pl.BlockSpec((1, 32), lambda bi: (0, 0)),
            pl.BlockSpec((9, 32, 64), lambda bi: (0, 0, 0)),
            pl.BlockSpec((1, 64), lambda bi: (0, 0)),
        ],
        out_specs=pl.BlockSpec((1, _H, _H, 64), lambda bi: (bi, 0, 0, 0)),
        out_shape=jax.ShapeDtypeStruct((_B, _H, _H, 64), jnp.float32),
    )(x16, w1, b1, w2, b2, w3, b3)


_DEC_TAPS = (
    ((0, 0, 0), (0, 1, 2), (1, 0, 6), (1, 1, 8)),
    ((0, 1, 1), (1, 1, 7)),
    ((1, 0, 3), (1, 1, 5)),
    ((1, 1, 4),),
)

_D2_ROW = (
    ((1, 0, 0), (0, 2, 1)),
    ((0, 1, 1),),
    ((0, 0, 1), (1, 2, 1)),
    ((1, 1, 1),),
)


def _dec_body(ze_ref, zq_ref, w1_ref, b1_ref, w2_ref, b2_ref, w3_ref, b3_ref,
              o_ref, zqo_ref):
    n = _H * _H
    ze = ze_ref[0]
    zq = zq_ref[0][:, :, :_D]
    zqo_ref[0] = zq
    x = ze + (zq - ze)
    xp = _pad_begin(x)
    acc = jnp.zeros((n, 128), jnp.float32)
    for t in range(4):
        dy, dx = divmod(t, 2)
        xs = xp[dy:dy + _H, dx:dx + _H, :].reshape(n, 64)
        acc = acc + jnp.dot(xs, w1_ref[t], preferred_element_type=jnp.float32)
    g = jnp.maximum(acc + b1_ref[0], 0.0).reshape(_H, _H, 128)
    gp = _pad_begin(g)
    acc = jnp.zeros((n, 256), jnp.float32)
    for t in range(4):
        dy, dx = divmod(t, 2)
        xs = gp[dy:dy + _H, dx:dx + _H, :].reshape(n, 128)
        acc = acc + jnp.dot(xs, w2_ref[t], preferred_element_type=jnp.float32)
    z2 = jnp.maximum(acc + b2_ref[0], 0.0).reshape(_H, _H, 256)
    z2p = jnp.pad(z2, ((1, 1), (1, 1), (0, 0)))
    acc = jnp.zeros((n, 16), jnp.float32)
    for t in range(9):
        sy, sx = divmod(t, 3)
        xs = z2p[sy:sy + _H, sx:sx + _H, :].reshape(n, 256)
        acc = acc + jnp.dot(xs, w3_ref[t], preferred_element_type=jnp.float32)
    o_ref[0] = (acc + b3_ref[0]).reshape(_H, _H, 16)


def _decoder(ze, zq, w1, b1, w2, b2, w3, b3):
    return pl.pallas_call(
        _dec_body,
        grid=(_B,),
        in_specs=[
            pl.BlockSpec((1, _H, _H, 64), lambda bi: (bi, 0, 0, 0)),
            pl.BlockSpec((1, _H, _H, 128), lambda bi: (bi, 0, 0, 0)),
            pl.BlockSpec((4, 64, 128), lambda bi: (0, 0, 0)),
            pl.BlockSpec((1, 128), lambda bi: (0, 0)),
            pl.BlockSpec((4, 128, 256), lambda bi: (0, 0, 0)),
            pl.BlockSpec((1, 256), lambda bi: (0, 0)),
            pl.BlockSpec((9, 256, 16), lambda bi: (0, 0, 0)),
            pl.BlockSpec((1, 16), lambda bi: (0, 0)),
        ],
        out_specs=[
            pl.BlockSpec((1, _H, _H, 16), lambda bi: (bi, 0, 0, 0)),
            pl.BlockSpec((1, _H, _H, _D), lambda bi: (bi, 0, 0, 0)),
        ],
        out_shape=[
            jax.ShapeDtypeStruct((_B, _H, _H, 16), jnp.float32),
            jax.ShapeDtypeStruct((_B, _H, _H, _D), jnp.float32),
        ],
    )(ze, zq, w1, b1, w2, b2, w3, b3)


_VQ_R = 1568
_ROWS = _B * _H * _H
_VQ_G = _ROWS // _VQ_R


def _vq_body(z_ref, cbt_ref, c2_ref, o_ref):
    z = z_ref[...]
    s = c2_ref[...] - 2.0 * jnp.dot(z, cbt_ref[...],
                                    preferred_element_type=jnp.float32)
    m = jnp.min(s, axis=1, keepdims=True)
    lane = lax.broadcasted_iota(jnp.int32, s.shape, 1)
    idx = jnp.min(jnp.where(s == m, lane, _K), axis=1)
    o_ref[0, 0] = idx


def _vq(zf, cbt, c2):
    out = pl.pallas_call(
        _vq_body,
        grid=(_VQ_G,),
        in_specs=[
            pl.BlockSpec((_VQ_R, _D), lambda i: (i, 0)),
            pl.BlockSpec((_D, _K), lambda i: (0, 0)),
            pl.BlockSpec((1, _K), lambda i: (0, 0)),
        ],
        out_specs=pl.BlockSpec((1, 1, _VQ_R), lambda i: (i, 0, 0)),
        out_shape=jax.ShapeDtypeStruct((_VQ_G, 1, _VQ_R), jnp.int32),
    )(zf, cbt, c2)
    return out.reshape(-1)


_NC, _NS = 2, 16
_NW = _NC * _NS
_RPW = _ROWS // _NW


def _sc_gather_body(idx_hbm, table_hbm, out_hbm, idx_v, rows_v, tbl_sh, sem):
    s = lax.axis_index("s")
    wid = s * _NC + lax.axis_index("c")
    base = wid * _RPW
    pltpu.sync_copy(idx_hbm.at[pl.ds(base, _RPW)], idx_v)

    @pl.when(s == 0)
    def _stage_table():
        pltpu.sync_copy(table_hbm, tbl_sh)

    plsc.subcore_barrier()
    pltpu.async_copy(tbl_sh.at[idx_v], rows_v, sem).wait()
    pltpu.sync_copy(rows_v, out_hbm.at[pl.ds(base, _RPW)])


def _sc_gather(idx, table128):
    mesh = plsc.VectorSubcoreMesh(core_axis_name="c", subcore_axis_name="s")
    f = functools.partial(
        pl.kernel,
        mesh=mesh,
        out_type=jax.ShapeDtypeStruct((_ROWS, 128), jnp.float32),
        scratch_types=[
            pltpu.VMEM((_RPW,), jnp.int32),
            pltpu.VMEM((_RPW, 128), jnp.float32),
            pltpu.VMEM_SHARED((_K, 128), jnp.float32),
            pltpu.SemaphoreType.DMA,
        ],
    )(_sc_gather_body)
    return f(idx, table128)


def _s2d16(x):
    s = (x.reshape(_B, _H, 4, _H, 4)
         .transpose(0, 1, 3, 2, 4)
         .reshape(_B, _H, _H, 16))
    return jnp.pad(s, ((0, 0), (0, 1), (0, 1), (0, 0)))


def _np_enc1_sel():
    T = np.zeros((2, 2, 16, 4, 9), np.float32)
    for p in range(2):
        for dy in range(3):
            oy, ry = divmod(2 * p + dy, 4)
            for q in range(2):
                for dx in range(3):
                    ox, rx = divmod(2 * q + dx, 4)
                    T[oy, ox, 4 * ry + rx, 2 * p + q, 3 * dy + dx] = 1.0
    return T.reshape(4, 16, 4, 9)


_ENC1_SEL = _np_enc1_sel()


def _enc1_weight(w):
    w1 = jnp.einsum('tiPk,kc->tiPc', _ENC1_SEL, w.reshape(9, 16))
    return w1.reshape(4, 16, 64)


def _np_dec1_sel():
    S = np.zeros((2, 2, 9, 4), np.float32)
    for p, taps in enumerate(_DEC_TAPS):
        for (dy, dx, k) in taps:
            S[dy, dx, k, p] = 1.0
    return S.reshape(4, 9, 4)


def _np_dec2_sel():
    U = np.zeros((2, 2, 4, 16, 9), np.float32)
    for ry in range(4):
        for (py, ky, oy) in _D2_ROW[ry]:
            for rx in range(4):
                for (px, kx, ox) in _D2_ROW[rx]:
                    U[oy, ox, 2 * py + px, 4 * ry + rx, 3 * ky + kx] = 1.0
    return U.reshape(4, 4, 16, 9)


def _np_dec3_sel():
    V = np.zeros((3, 3, 16, 16, 9), np.float32)
    for ry in range(4):
        for ky in range(3):
            sy, py = divmod(ry + ky - 1, 4)
            for rx in range(4):
                for kx in range(3):
                    sx, px = divmod(rx + kx - 1, 4)
                    V[sy + 1, sx + 1, 4 * py + px, 4 * ry + rx,
                      3 * ky + kx] = 1.0
    return V.reshape(9, 16, 16, 9)


_DEC1_SEL = _np_dec1_sel()
_DEC2_SEL = _np_dec2_sel()
_DEC3_SEL = _np_dec3_sel()


def _dec1_weight(w):
    C = jnp.einsum('tkp,kcd->tcpd', _DEC1_SEL, w.reshape(9, 64, 32))
    return C.reshape(4, 64, 128)


def _dec2_weight(w):
    Bm = jnp.einsum('tiok,kcd->ticod', _DEC2_SEL, w.reshape(9, 32, 16))
    return Bm.reshape(4, 128, 256)


def _dec3_weight(w):
    A = jnp.einsum('siok,kc->sico', _DEC3_SEL, w.reshape(9, 16))
    return A.reshape(9, 256, 16)


def _s2d_weight(w):
    _, _, ci, co = w.shape
    wp = jnp.pad(w, ((0, 1), (0, 1), (0, 0), (0, 0)))
    w4 = wp.reshape(2, 2, 2, 2, ci, co).transpose(0, 2, 1, 3, 4, 5)
    return w4.reshape(4, 4 * ci, co)


def kernel(inputs, enc_w1, enc_b1, enc_w2, enc_b2, enc_w3, enc_b3, codebook,
           dec_w1, dec_b1, dec_w2, dec_b2, dec_w3, dec_b3):
    x16 = _s2d16(inputs)
    z_e = _encoder(x16, _enc1_weight(enc_w1), jnp.tile(enc_b1, 4)[None],
                   _s2d_weight(enc_w2), enc_b2[None],
                   enc_w3.reshape(9, 32, 64), enc_b3[None])

    zf = z_e.reshape(_ROWS, _D)
    cbt = codebook.T
    c2 = jnp.sum(codebook * codebook, axis=1)[None, :]
    qzf = _vq(zf, cbt, c2)
    q_z = qzf.reshape(_B, _H, _H)
    cb128 = jnp.pad(codebook, ((0, 0), (0, 128 - _D)))
    zq_f = _sc_gather(qzf, cb128)
    zq128 = zq_f.reshape(_B, _H, _H, 128)

    ph, z_q = _decoder(z_e, zq128, _dec1_weight(dec_w1),
                       jnp.tile(dec_b1, 4)[None],
                       _dec2_weight(dec_w2), jnp.tile(dec_b2, 16)[None],
                       _dec3_weight(dec_w3), jnp.tile(dec_b3, 16)[None])
    logits = (ph.reshape(_B, _H, _H, 4, 4)
              .transpose(0, 1, 3, 2, 4)
              .reshape(_B, 224, 224, 1))
    return (logits, z_e, z_q, q_z)


# speedup vs baseline: 2.6470x; 1.0377x over previous
import functools

import jax
import jax.numpy as jnp
import numpy as np
from jax import lax
from jax.experimental import pallas as pl
from jax.experimental.pallas import tpu as pltpu
from jax.experimental.pallas import tpu_sc as plsc

_B = 8
_D = 64
_K = 1024
_H = 56


def _pad_end(x, n=1):
    return jnp.pad(x, ((0, n), (0, n), (0, 0)))


def _pad_begin(x, n=1):
    return jnp.pad(x, ((n, 0), (n, 0), (0, 0)))


def _enc_body(x_ref, w1_ref, b1_ref, w2_ref, b2_ref, w3_ref, b3_ref,
              cbt_ref, c2_ref, o_ref, qz_ref):
    n = _H * _H
    x = x_ref[0]
    acc = jnp.zeros((n, 64), jnp.float32)
    for t, (oy, ox) in enumerate(((0, 0), (0, 1), (1, 0), (1, 1))):
        xs = x[oy:oy + _H, ox:ox + _H, :].reshape(n, 16)
        acc = acc + jnp.dot(xs, w1_ref[t], preferred_element_type=jnp.float32)
    s1 = jnp.maximum(acc + b1_ref[0], 0.0).reshape(_H, _H, 64)
    s1p = _pad_end(s1)
    acc = jnp.zeros((n, 32), jnp.float32)
    for t, (oy, ox) in enumerate(((0, 0), (0, 1), (1, 0), (1, 1))):
        xs = s1p[oy:oy + _H, ox:ox + _H, :].reshape(n, 64)
        acc = acc + jnp.dot(xs, w2_ref[t], preferred_element_type=jnp.float32)
    s2 = jnp.maximum(acc + b2_ref[0], 0.0).reshape(_H, _H, 32)
    s2p = jnp.pad(s2, ((1, 1), (1, 1), (0, 0)))
    acc = jnp.zeros((n, 64), jnp.float32)
    for t in range(9):
        dy, dx = divmod(t, 3)
        xs = s2p[dy:dy + _H, dx:dx + _H, :].reshape(n, 32)
        acc = acc + jnp.dot(xs, w3_ref[t], preferred_element_type=jnp.float32)
    ze = acc + b3_ref[0]
    o_ref[0] = ze.reshape(_H, _H, 64)
    nc = n // 4
    for c in range(4):
        zc = ze[c * nc:(c + 1) * nc]
        s = c2_ref[...] - 2.0 * jnp.dot(zc, cbt_ref[...],
                                        preferred_element_type=jnp.float32)
        m = jnp.min(s, axis=1, keepdims=True)
        lane = lax.broadcasted_iota(jnp.int32, s.shape, 1)
        qz_ref[0, 0, c * nc:(c + 1) * nc] = jnp.min(
            jnp.where(s == m, lane, _K), axis=1)


def _encoder(x16, w1, b1, w2, b2, w3, b3, cbt, c2):
    return pl.pallas_call(
        _enc_body,
        grid=(_B,),
        in_specs=[
            pl.BlockSpec((1, 57, 57, 16), lambda bi: (bi, 0, 0, 0)),
            pl.BlockSpec((4, 16, 64), lambda bi: (0, 0, 0)),
            pl.BlockSpec((1, 64), lambda bi: (0, 0)),
            pl.BlockSpec((4, 64, 32), lambda bi: (0, 0, 0)),
            pl.BlockSpec((1, 32), lambda bi: (0, 0)),
            pl.BlockSpec((9, 32, 64), lambda bi: (0, 0, 0)),
            pl.BlockSpec((1, 64), lambda bi: (0, 0)),
            pl.BlockSpec((_D, _K), lambda bi: (0, 0)),
            pl.BlockSpec((1, _K), lambda bi: (0, 0)),
        ],
        out_specs=[
            pl.BlockSpec((1, _H, _H, 64), lambda bi: (bi, 0, 0, 0)),
            pl.BlockSpec((1, 1, _H * _H), lambda bi: (bi, 0, 0)),
        ],
        out_shape=[
            jax.ShapeDtypeStruct((_B, _H, _H, 64), jnp.float32),
            jax.ShapeDtypeStruct((_B, 1, _H * _H), jnp.int32),
        ],
    )(x16, w1, b1, w2, b2, w3, b3, cbt, c2)


_DEC_TAPS = (
    ((0, 0, 0), (0, 1, 2), (1, 0, 6), (1, 1, 8)),
    ((0, 1, 1), (1, 1, 7)),
    ((1, 0, 3), (1, 1, 5)),
    ((1, 1, 4),),
)

_D2_ROW = (
    ((1, 0, 0), (0, 2, 1)),
    ((0, 1, 1),),
    ((0, 0, 1), (1, 2, 1)),
    ((1, 1, 1),),
)


def _dec_body(ze_ref, zq_ref, w1_ref, b1_ref, w2_ref, b2_ref, w3_ref, b3_ref,
              o_ref, zqo_ref):
    n = _H * _H
    ze = ze_ref[0]
    zq = zq_ref[0][:, :, :_D]
    zqo_ref[0] = zq
    x = ze + (zq - ze)
    xp = _pad_begin(x)
    acc = jnp.zeros((n, 128), jnp.float32)
    for t in range(4):
        dy, dx = divmod(t, 2)
        xs = xp[dy:dy + _H, dx:dx + _H, :].reshape(n, 64)
        acc = acc + jnp.dot(xs, w1_ref[t], preferred_element_type=jnp.float32)
    g = jnp.maximum(acc + b1_ref[0], 0.0).reshape(_H, _H, 128)
    gp = _pad_begin(g)
    acc = jnp.zeros((n, 256), jnp.float32)
    for t in range(4):
        dy, dx = divmod(t, 2)
        xs = gp[dy:dy + _H, dx:dx + _H, :].reshape(n, 128)
        acc = acc + jnp.dot(xs, w2_ref[t], preferred_element_type=jnp.float32)
    z2 = jnp.maximum(acc + b2_ref[0], 0.0).reshape(_H, _H, 256)
    z2p = jnp.pad(z2, ((1, 1), (1, 1), (0, 0)))
    acc = jnp.zeros((n, 16), jnp.float32)
    for t in range(9):
        sy, sx = divmod(t, 3)
        xs = z2p[sy:sy + _H, sx:sx + _H, :].reshape(n, 256)
        acc = acc + jnp.dot(xs, w3_ref[t], preferred_element_type=jnp.float32)
    o_ref[0] = (acc + b3_ref[0]).reshape(_H, _H, 16)


def _decoder(ze, zq, w1, b1, w2, b2, w3, b3):
    return pl.pallas_call(
        _dec_body,
        grid=(_B,),
        in_specs=[
            pl.BlockSpec((1, _H, _H, 64), lambda bi: (bi, 0, 0, 0)),
            pl.BlockSpec((1, _H, _H, 128), lambda bi: (bi, 0, 0, 0)),
            pl.BlockSpec((4, 64, 128), lambda bi: (0, 0, 0)),
            pl.BlockSpec((1, 128), lambda bi: (0, 0)),
            pl.BlockSpec((4, 128, 256), lambda bi: (0, 0, 0)),
            pl.BlockSpec((1, 256), lambda bi: (0, 0)),
            pl.BlockSpec((9, 256, 16), lambda bi: (0, 0, 0)),
            pl.BlockSpec((1, 16), lambda bi: (0, 0)),
        ],
        out_specs=[
            pl.BlockSpec((1, _H, _H, 16), lambda bi: (bi, 0, 0, 0)),
            pl.BlockSpec((1, _H, _H, _D), lambda bi: (bi, 0, 0, 0)),
        ],
        out_shape=[
            jax.ShapeDtypeStruct((_B, _H, _H, 16), jnp.float32),
            jax.ShapeDtypeStruct((_B, _H, _H, _D), jnp.float32),
        ],
    )(ze, zq, w1, b1, w2, b2, w3, b3)


_ROWS = _B * _H * _H


_NC, _NS = 2, 16
_NW = _NC * _NS
_RPW = _ROWS // _NW


def _sc_gather_body(idx_hbm, table_hbm, out_hbm, idx_v, rows_v, tbl_sh, sem):
    s = lax.axis_index("s")
    wid = s * _NC + lax.axis_index("c")
    base = wid * _RPW
    pltpu.sync_copy(idx_hbm.at[pl.ds(base, _RPW)], idx_v)

    @pl.when(s == 0)
    def _stage_table():
        pltpu.sync_copy(table_hbm, tbl_sh)

    plsc.subcore_barrier()
    pltpu.async_copy(tbl_sh.at[idx_v], rows_v, sem).wait()
    pltpu.sync_copy(rows_v, out_hbm.at[pl.ds(base, _RPW)])


def _sc_gather(idx, table128):
    mesh = plsc.VectorSubcoreMesh(core_axis_name="c", subcore_axis_name="s")
    f = functools.partial(
        pl.kernel,
        mesh=mesh,
        out_type=jax.ShapeDtypeStruct((_ROWS, 128), jnp.float32),
        scratch_types=[
            pltpu.VMEM((_RPW,), jnp.int32),
            pltpu.VMEM((_RPW, 128), jnp.float32),
            pltpu.VMEM_SHARED((_K, 128), jnp.float32),
            pltpu.SemaphoreType.DMA,
        ],
    )(_sc_gather_body)
    return f(idx, table128)


def _s2d16(x):
    s = (x.reshape(_B, _H, 4, _H, 4)
         .transpose(0, 1, 3, 2, 4)
         .reshape(_B, _H, _H, 16))
    return jnp.pad(s, ((0, 0), (0, 1), (0, 1), (0, 0)))


def _np_enc1_sel():
    T = np.zeros((2, 2, 16, 4, 9), np.float32)
    for p in range(2):
        for dy in range(3):
            oy, ry = divmod(2 * p + dy, 4)
            for q in range(2):
                for dx in range(3):
                    ox, rx = divmod(2 * q + dx, 4)
                    T[oy, ox, 4 * ry + rx, 2 * p + q, 3 * dy + dx] = 1.0
    return T.reshape(4, 16, 4, 9)


_ENC1_SEL = _np_enc1_sel()


def _enc1_weight(w):
    w1 = jnp.einsum('tiPk,kc->tiPc', _ENC1_SEL, w.reshape(9, 16))
    return w1.reshape(4, 16, 64)


def _np_dec1_sel():
    S = np.zeros((2, 2, 9, 4), np.float32)
    for p, taps in enumerate(_DEC_TAPS):
        for (dy, dx, k) in taps:
            S[dy, dx, k, p] = 1.0
    return S.reshape(4, 9, 4)


def _np_dec2_sel():
    U = np.zeros((2, 2, 4, 16, 9), np.float32)
    for ry in range(4):
        for (py, ky, oy) in _D2_ROW[ry]:
            for rx in range(4):
                for (px, kx, ox) in _D2_ROW[rx]:
                    U[oy, ox, 2 * py + px, 4 * ry + rx, 3 * ky + kx] = 1.0
    return U.reshape(4, 4, 16, 9)


def _np_dec3_sel():
    V = np.zeros((3, 3, 16, 16, 9), np.float32)
    for ry in range(4):
        for ky in range(3):
            sy, py = divmod(ry + ky - 1, 4)
            for rx in range(4):
                for kx in range(3):
                    sx, px = divmod(rx + kx - 1, 4)
                    V[sy + 1, sx + 1, 4 * py + px, 4 * ry + rx,
                      3 * ky + kx] = 1.0
    return V.reshape(9, 16, 16, 9)


_DEC1_SEL = _np_dec1_sel()
_DEC2_SEL = _np_dec2_sel()
_DEC3_SEL = _np_dec3_sel()


def _dec1_weight(w):
    C = jnp.einsum('tkp,kcd->tcpd', _DEC1_SEL, w.reshape(9, 64, 32))
    return C.reshape(4, 64, 128)


def _dec2_weight(w):
    Bm = jnp.einsum('tiok,kcd->ticod', _DEC2_SEL, w.reshape(9, 32, 16))
    return Bm.reshape(4, 128, 256)


def _dec3_weight(w):
    A = jnp.einsum('siok,kc->sico', _DEC3_SEL, w.reshape(9, 16))
    return A.reshape(9, 256, 16)


def _s2d_weight(w):
    _, _, ci, co = w.shape
    wp = jnp.pad(w, ((0, 1), (0, 1), (0, 0), (0, 0)))
    w4 = wp.reshape(2, 2, 2, 2, ci, co).transpose(0, 2, 1, 3, 4, 5)
    return w4.reshape(4, 4 * ci, co)


def kernel(inputs, enc_w1, enc_b1, enc_w2, enc_b2, enc_w3, enc_b3, codebook,
           dec_w1, dec_b1, dec_w2, dec_b2, dec_w3, dec_b3):
    x16 = _s2d16(inputs)
    cbt = codebook.T
    c2 = jnp.sum(codebook * codebook, axis=1)[None, :]
    z_e, qz = _encoder(x16, _enc1_weight(enc_w1), jnp.tile(enc_b1, 4)[None],
                       _s2d_weight(enc_w2), enc_b2[None],
                       enc_w3.reshape(9, 32, 64), enc_b3[None], cbt, c2)
    qzf = qz.reshape(_ROWS)
    q_z = qzf.reshape(_B, _H, _H)
    cb128 = jnp.pad(codebook, ((0, 0), (0, 128 - _D)))
    zq_f = _sc_gather(qzf, cb128)
    zq128 = zq_f.reshape(_B, _H, _H, 128)

    ph, z_q = _decoder(z_e, zq128, _dec1_weight(dec_w1),
                       jnp.tile(dec_b1, 4)[None],
                       _dec2_weight(dec_w2), jnp.tile(dec_b2, 16)[None],
                       _dec3_weight(dec_w3), jnp.tile(dec_b3, 16)[None])
    logits = (ph.reshape(_B, _H, _H, 4, 4)
              .transpose(0, 1, 3, 2, 4)
              .reshape(_B, 224, 224, 1))
    return (logits, z_e, z_q, q_z)


# speedup vs baseline: 2.7530x; 1.0400x over previous
import functools

import jax
import jax.numpy as jnp
import numpy as np
from jax import lax
from jax.experimental import pallas as pl
from jax.experimental.pallas import tpu as pltpu
from jax.experimental.pallas import tpu_sc as plsc

_B = 8
_D = 64
_K = 1024
_H = 56


def _pad_end(x, n=1):
    return jnp.pad(x, ((0, n), (0, n), (0, 0)))


def _pad_begin(x, n=1):
    return jnp.pad(x, ((n, 0), (n, 0), (0, 0)))


def _enc_body(x_ref, w1_ref, b1_ref, w2_ref, b2_ref, w3_ref, b3_ref,
              cbt_ref, c2_ref, o_ref, qz_ref):
    n = _H * _H
    x = _pad_end(x_ref[0])
    acc = jnp.zeros((n, 64), jnp.float32)
    for t, (oy, ox) in enumerate(((0, 0), (0, 1), (1, 0), (1, 1))):
        xs = x[oy:oy + _H, ox:ox + _H, :].reshape(n, 16)
        acc = acc + jnp.dot(xs, w1_ref[t], preferred_element_type=jnp.float32)
    s1 = jnp.maximum(acc + b1_ref[0], 0.0).reshape(_H, _H, 64)
    s1p = _pad_end(s1)
    acc = jnp.zeros((n, 32), jnp.float32)
    for t, (oy, ox) in enumerate(((0, 0), (0, 1), (1, 0), (1, 1))):
        xs = s1p[oy:oy + _H, ox:ox + _H, :].reshape(n, 64)
        acc = acc + jnp.dot(xs, w2_ref[t], preferred_element_type=jnp.float32)
    s2 = jnp.maximum(acc + b2_ref[0], 0.0).reshape(_H, _H, 32)
    s2p = jnp.pad(s2, ((1, 1), (1, 1), (0, 0)))
    acc = jnp.zeros((n, 64), jnp.float32)
    for t in range(9):
        dy, dx = divmod(t, 3)
        xs = s2p[dy:dy + _H, dx:dx + _H, :].reshape(n, 32)
        acc = acc + jnp.dot(xs, w3_ref[t], preferred_element_type=jnp.float32)
    ze = acc + b3_ref[0]
    o_ref[0] = ze.reshape(_H, _H, 64)
    nc = n // 4
    for c in range(4):
        zc = ze[c * nc:(c + 1) * nc]
        s = c2_ref[...] - 2.0 * jnp.dot(zc, cbt_ref[...],
                                        preferred_element_type=jnp.float32)
        m = jnp.min(s, axis=1, keepdims=True)
        lane = lax.broadcasted_iota(jnp.int32, s.shape, 1)
        qz_ref[0, 0, c * nc:(c + 1) * nc] = jnp.min(
            jnp.where(s == m, lane, _K), axis=1)


def _encoder(x16, w1, b1, w2, b2, w3, b3, cbt, c2):
    return pl.pallas_call(
        _enc_body,
        grid=(_B,),
        in_specs=[
            pl.BlockSpec((1, _H, _H, 16), lambda bi: (bi, 0, 0, 0)),
            pl.BlockSpec((4, 16, 64), lambda bi: (0, 0, 0)),
            pl.BlockSpec((1, 64), lambda bi: (0, 0)),
            pl.BlockSpec((4, 64, 32), lambda bi: (0, 0, 0)),
            pl.BlockSpec((1, 32), lambda bi: (0, 0)),
            pl.BlockSpec((9, 32, 64), lambda bi: (0, 0, 0)),
            pl.BlockSpec((1, 64), lambda bi: (0, 0)),
            pl.BlockSpec((_D, _K), lambda bi: (0, 0)),
            pl.BlockSpec((1, _K), lambda bi: (0, 0)),
        ],
        out_specs=[
            pl.BlockSpec((1, _H, _H, 64), lambda bi: (bi, 0, 0, 0)),
            pl.BlockSpec((1, 1, _H * _H), lambda bi: (bi, 0, 0)),
        ],
        out_shape=[
            jax.ShapeDtypeStruct((_B, _H, _H, 64), jnp.float32),
            jax.ShapeDtypeStruct((_B, 1, _H * _H), jnp.int32),
        ],
    )(x16, w1, b1, w2, b2, w3, b3, cbt, c2)


_DEC_TAPS = (
    ((0, 0, 0), (0, 1, 2), (1, 0, 6), (1, 1, 8)),
    ((0, 1, 1), (1, 1, 7)),
    ((1, 0, 3), (1, 1, 5)),
    ((1, 1, 4),),
)

_D2_ROW = (
    ((1, 0, 0), (0, 2, 1)),
    ((0, 1, 1),),
    ((0, 0, 1), (1, 2, 1)),
    ((1, 1, 1),),
)


def _dec_body(ze_ref, zq_ref, w1_ref, b1_ref, w2_ref, b2_ref, w3_ref, b3_ref,
              o_ref, zqo_ref):
    n = _H * _H
    ze = ze_ref[0]
    zq = zq_ref[0][:, :, :_D]
    zqo_ref[0] = zq
    x = ze + (zq - ze)
    xp = _pad_begin(x)
    acc = jnp.zeros((n, 128), jnp.float32)
    for t in range(4):
        dy, dx = divmod(t, 2)
        xs = xp[dy:dy + _H, dx:dx + _H, :].reshape(n, 64)
        acc = acc + jnp.dot(xs, w1_ref[t], preferred_element_type=jnp.float32)
    g = jnp.maximum(acc + b1_ref[0], 0.0).reshape(_H, _H, 128)
    gp = _pad_begin(g)
    acc = jnp.zeros((n, 256), jnp.float32)
    for t in range(4):
        dy, dx = divmod(t, 2)
        xs = gp[dy:dy + _H, dx:dx + _H, :].reshape(n, 128)
        acc = acc + jnp.dot(xs, w2_ref[t], preferred_element_type=jnp.float32)
    z2 = jnp.maximum(acc + b2_ref[0], 0.0).reshape(_H, _H, 256)
    z2p = jnp.pad(z2, ((1, 1), (1, 1), (0, 0)))
    acc = jnp.zeros((n, 16), jnp.float32)
    for t in range(9):
        sy, sx = divmod(t, 3)
        xs = z2p[sy:sy + _H, sx:sx + _H, :].reshape(n, 256)
        acc = acc + jnp.dot(xs, w3_ref[t], preferred_element_type=jnp.float32)
    o_ref[0] = (acc + b3_ref[0]).reshape(_H, _H, 16)


def _decoder(ze, zq, w1, b1, w2, b2, w3, b3):
    return pl.pallas_call(
        _dec_body,
        grid=(_B,),
        in_specs=[
            pl.BlockSpec((1, _H, _H, 64), lambda bi: (bi, 0, 0, 0)),
            pl.BlockSpec((1, _H, _H, 128), lambda bi: (bi, 0, 0, 0)),
            pl.BlockSpec((4, 64, 128), lambda bi: (0, 0, 0)),
            pl.BlockSpec((1, 128), lambda bi: (0, 0)),
            pl.BlockSpec((4, 128, 256), lambda bi: (0, 0, 0)),
            pl.BlockSpec((1, 256), lambda bi: (0, 0)),
            pl.BlockSpec((9, 256, 16), lambda bi: (0, 0, 0)),
            pl.BlockSpec((1, 16), lambda bi: (0, 0)),
        ],
        out_specs=[
            pl.BlockSpec((1, _H, _H, 16), lambda bi: (bi, 0, 0, 0)),
            pl.BlockSpec((1, _H, _H, _D), lambda bi: (bi, 0, 0, 0)),
        ],
        out_shape=[
            jax.ShapeDtypeStruct((_B, _H, _H, 16), jnp.float32),
            jax.ShapeDtypeStruct((_B, _H, _H, _D), jnp.float32),
        ],
    )(ze, zq, w1, b1, w2, b2, w3, b3)


_ROWS = _B * _H * _H


_NC, _NS = 2, 16
_NW = _NC * _NS
_RPW = _ROWS // _NW


def _sc_gather_body(idx_hbm, table_hbm, out_hbm, idx_v, rows_v, tbl_sh, sem):
    s = lax.axis_index("s")
    wid = s * _NC + lax.axis_index("c")
    base = wid * _RPW
    pltpu.sync_copy(idx_hbm.at[pl.ds(base, _RPW)], idx_v)

    @pl.when(s == 0)
    def _stage_table():
        pltpu.sync_copy(table_hbm, tbl_sh)

    plsc.subcore_barrier()
    pltpu.async_copy(tbl_sh.at[idx_v], rows_v, sem).wait()
    pltpu.sync_copy(rows_v, out_hbm.at[pl.ds(base, _RPW)])


def _sc_gather(idx, table128):
    mesh = plsc.VectorSubcoreMesh(core_axis_name="c", subcore_axis_name="s")
    f = functools.partial(
        pl.kernel,
        mesh=mesh,
        out_type=jax.ShapeDtypeStruct((_ROWS, 128), jnp.float32),
        scratch_types=[
            pltpu.VMEM((_RPW,), jnp.int32),
            pltpu.VMEM((_RPW, 128), jnp.float32),
            pltpu.VMEM_SHARED((_K, 128), jnp.float32),
            pltpu.SemaphoreType.DMA,
        ],
    )(_sc_gather_body)
    return f(idx, table128)


def _s2d16(x):
    return (x.reshape(_B, _H, 4, _H, 4)
            .transpose(0, 1, 3, 2, 4)
            .reshape(_B, _H, _H, 16))


def _np_enc1_sel():
    T = np.zeros((2, 2, 16, 4, 9), np.float32)
    for p in range(2):
        for dy in range(3):
            oy, ry = divmod(2 * p + dy, 4)
            for q in range(2):
                for dx in range(3):
                    ox, rx = divmod(2 * q + dx, 4)
                    T[oy, ox, 4 * ry + rx, 2 * p + q, 3 * dy + dx] = 1.0
    return T.reshape(4, 16, 4, 9)


_ENC1_SEL = _np_enc1_sel()


def _enc1_weight(w):
    w1 = jnp.einsum('tiPk,kc->tiPc', _ENC1_SEL, w.reshape(9, 16))
    return w1.reshape(4, 16, 64)


def _np_dec1_sel():
    S = np.zeros((2, 2, 9, 4), np.float32)
    for p, taps in enumerate(_DEC_TAPS):
        for (dy, dx, k) in taps:
            S[dy, dx, k, p] = 1.0
    return S.reshape(4, 9, 4)


def _np_dec2_sel():
    U = np.zeros((2, 2, 4, 16, 9), np.float32)
    for ry in range(4):
        for (py, ky, oy) in _D2_ROW[ry]:
            for rx in range(4):
                for (px, kx, ox) in _D2_ROW[rx]:
                    U[oy, ox, 2 * py + px, 4 * ry + rx, 3 * ky + kx] = 1.0
    return U.reshape(4, 4, 16, 9)


def _np_dec3_sel():
    V = np.zeros((3, 3, 16, 16, 9), np.float32)
    for ry in range(4):
        for ky in range(3):
            sy, py = divmod(ry + ky - 1, 4)
            for rx in range(4):
                for kx in range(3):
                    sx, px = divmod(rx + kx - 1, 4)
                    V[sy + 1, sx + 1, 4 * py + px, 4 * ry + rx,
                      3 * ky + kx] = 1.0
    return V.reshape(9, 16, 16, 9)


_DEC1_SEL = _np_dec1_sel()
_DEC2_SEL = _np_dec2_sel()
_DEC3_SEL = _np_dec3_sel()


def _dec1_weight(w):
    C = jnp.einsum('tkp,kcd->tcpd', _DEC1_SEL, w.reshape(9, 64, 32))
    return C.reshape(4, 64, 128)


def _dec2_weight(w):
    Bm = jnp.einsum('tiok,kcd->ticod', _DEC2_SEL, w.reshape(9, 32, 16))
    return Bm.reshape(4, 128, 256)


def _dec3_weight(w):
    A = jnp.einsum('siok,kc->sico', _DEC3_SEL, w.reshape(9, 16))
    return A.reshape(9, 256, 16)


def _s2d_weight(w):
    _, _, ci, co = w.shape
    wp = jnp.pad(w, ((0, 1), (0, 1), (0, 0), (0, 0)))
    w4 = wp.reshape(2, 2, 2, 2, ci, co).transpose(0, 2, 1, 3, 4, 5)
    return w4.reshape(4, 4 * ci, co)


def kernel(inputs, enc_w1, enc_b1, enc_w2, enc_b2, enc_w3, enc_b3, codebook,
           dec_w1, dec_b1, dec_w2, dec_b2, dec_w3, dec_b3):
    x16 = _s2d16(inputs)
    cbt = codebook.T
    c2 = jnp.sum(codebook * codebook, axis=1)[None, :]
    z_e, qz = _encoder(x16, _enc1_weight(enc_w1), jnp.tile(enc_b1, 4)[None],
                       _s2d_weight(enc_w2), enc_b2[None],
                       enc_w3.reshape(9, 32, 64), enc_b3[None], cbt, c2)
    qzf = qz.reshape(_ROWS)
    q_z = qzf.reshape(_B, _H, _H)
    cb128 = jnp.pad(codebook, ((0, 0), (0, 128 - _D)))
    zq_f = _sc_gather(qzf, cb128)
    zq128 = zq_f.reshape(_B, _H, _H, 128)

    ph, z_q = _decoder(z_e, zq128, _dec1_weight(dec_w1),
                       jnp.tile(dec_b1, 4)[None],
                       _dec2_weight(dec_w2), jnp.tile(dec_b2, 16)[None],
                       _dec3_weight(dec_w3), jnp.tile(dec_b3, 16)[None])
    logits = (ph.reshape(_B, _H, _H, 4, 4)
              .transpose(0, 1, 3, 2, 4)
              .reshape(_B, 224, 224, 1))
    return (logits, z_e, z_q, q_z)
